# Initial kernel scaffold; baseline (speedup 1.0000x reference)
import jax
import jax.numpy as jnp
from jax.experimental import pallas as pl


def kernel(x, edge_index, edge_attr, batch, params):
    raise NotImplementedError("write your pallas kernel here")


# speedup vs baseline: 1.0472x; 1.0472x over previous
import functools

import jax
import jax.numpy as jnp
from jax import lax
from jax.experimental import pallas as pl
from jax.experimental.pallas import tpu as pltpu
from jax.experimental.pallas import tpu_sc as plsc

F32 = jnp.float32
HID = 96
CP = 128
NB = 64
CHUNK = 48
SBLK = 16
TPC = 16
NRANGE = 4
NPAD = 51200
RNG = NPAD // NRANGE
EBLK = SBLK * CHUNK
NBLK = -(-800000 // (TPC * EBLK))
EPT = EBLK * NBLK
EPAD = TPC * EPT


def _emb_body(r, x_ref, emb_ref, out_ref):
    ids = x_ref[0]
    oh = (ids == lax.broadcasted_iota(jnp.int32, (r, 128), 1)).astype(F32)
    out_ref[...] = jnp.dot(oh, emb_ref[...], preferred_element_type=F32)


def _edge_body(ea_ref, w1_ref, b1_ref, w2_ref, b2_ref, o0, o1, o2, o3):
    ea = ea_ref[...]
    outs = (o0, o1, o2, o3)
    for l in range(4):
        z = jnp.dot(ea, w1_ref[l], preferred_element_type=F32) + b1_ref[l]
        outs[l][...] = jnp.dot(jax.nn.softplus(z), w2_ref[l],
                               preferred_element_type=F32) + b2_ref[l]


def _node_body(agg_ref, w_ref, b_ref, out_ref):
    z = jnp.dot(agg_ref[...], w_ref[...], preferred_element_type=F32) + b_ref[...]
    out_ref[...] = jax.nn.softplus(z)


def _node_pool_body(bn, agg_ref, w_ref, b_ref, bat_ref, sum_ref, cnt_ref):
    i = pl.program_id(0)
    z = jnp.dot(agg_ref[...], w_ref[...], preferred_element_type=F32) + b_ref[...]
    h = jax.nn.softplus(z)
    ids = bat_ref[0]
    oh = (ids == lax.broadcasted_iota(jnp.int32, (bn, NB), 1)).astype(F32)
    ps = lax.dot_general(oh, h, (((0,), (0,)), ((), ())),
                         preferred_element_type=F32)
    pc = lax.dot_general(oh, jnp.ones_like(h), (((0,), (0,)), ((), ())),
                         preferred_element_type=F32)

    @pl.when(i == 0)
    def _():
        sum_ref[...] = jnp.zeros_like(sum_ref)
        cnt_ref[...] = jnp.zeros_like(cnt_ref)

    sum_ref[...] += ps
    cnt_ref[...] += pc


def _reg_body(sum_ref, cnt_ref, w1, b1, w2, b2, w3, b3, out_ref):
    mean = sum_ref[...] / jnp.maximum(cnt_ref[...], 1.0)
    r = jax.nn.softplus(jnp.dot(mean, w1[...], preferred_element_type=F32) + b1[...])
    r = jax.nn.softplus(jnp.dot(r, w2[...], preferred_element_type=F32) + b2[...])
    out_ref[...] = jnp.dot(r, w3[...], preferred_element_type=F32) + b3[...]


def _mp_body(dst_hbm, src_hbm, ew_hbm, h_hbm, out_hbm,
             dstg, srcg, dstidx, hbufa, ewbufa, hbufb, ewbufb,
             aggsp, sem_ha, sem_ea, sem_hb, sem_eb):
    c = lax.axis_index("c")
    s = lax.axis_index("s")
    zrows = RNG // TPC
    zero16 = jnp.zeros((16,), F32)

    def zb(i, carry):
        for j in range(CP // 16):
            hbufa[i, pl.ds(j * 16, 16)] = zero16
        return carry

    lax.fori_loop(0, CHUNK, zb, 0)

    tilebase = s * EPT

    for r in range(2):
        lo = (2 * c + r) * RNG
        trash = RNG + s
        for k in range(zrows // CHUNK):
            pltpu.sync_copy(hbufa,
                            aggsp.at[pl.ds(s * zrows + k * CHUNK, CHUNK)])
        zrem = zrows - (zrows // CHUNK) * CHUNK
        if zrem:
            pltpu.sync_copy(
                hbufa.at[pl.ds(0, zrem)],
                aggsp.at[pl.ds(s * zrows + (zrows // CHUNK) * CHUNK, zrem)])
        plsc.subcore_barrier()

        def block_body(b, carry):
            bbase = tilebase + b * EBLK
            pltpu.sync_copy(dst_hbm.at[pl.ds(bbase, EBLK)], dstg)
            pltpu.sync_copy(src_hbm.at[pl.ds(bbase, EBLK)], srcg)

            def issue(k2, hb, eb, sh, se):
                off = k2 * CHUNK
                cph = pltpu.async_copy(
                    h_hbm.at[srcg.at[pl.ds(off, CHUNK)]], hb, sh)
                cpe = pltpu.async_copy(
                    ew_hbm.at[pl.ds(bbase + off, CHUNK)], eb, se)
                return cph, cpe

            def process(k2, hb, eb):
                def cpidx(t, carry3):
                    sl = pl.ds(t * 16, 16)
                    d16 = dstg[pl.ds(k2 * CHUNK + t * 16, 16)] - lo
                    m = (d16 >= 0) & (d16 < RNG)
                    dstidx[sl] = jnp.where(m, d16, trash)
                    return carry3

                lax.fori_loop(0, CHUNK // 16, cpidx, 0)

                def mul(i, carry3):
                    for j in range(CP // 16):
                        sl = pl.ds(j * 16, 16)
                        hb[i, sl] = hb[i, sl] * eb[i, sl]
                    return carry3

                lax.fori_loop(0, CHUNK, mul, 0)
                pltpu.sync_copy(hb, aggsp.at[dstidx], add=True)

            da = issue(0, hbufa, ewbufa, sem_ha, sem_ea)
            for q in range(SBLK // 2):
                db = issue(2 * q + 1, hbufb, ewbufb, sem_hb, sem_eb)
                da[0].wait()
                da[1].wait()
                process(2 * q, hbufa, ewbufa)
                if 2 * q + 2 < SBLK:
                    da = issue(2 * q + 2, hbufa, ewbufa, sem_ha, sem_ea)
                db[0].wait()
                db[1].wait()
                process(2 * q + 1, hbufb, ewbufb)
            return carry

        lax.fori_loop(0, NBLK, block_body, 0)
        plsc.subcore_barrier()
        pltpu.sync_copy(aggsp.at[pl.ds(s * zrows, zrows)],
                        out_hbm.at[pl.ds(lo + s * zrows, zrows)])
        plsc.subcore_barrier()

        if r == 0:
            lax.fori_loop(0, CHUNK, zb, 0)


def _message_pass(dst1, src1, ew, h):
    mesh = plsc.VectorSubcoreMesh(core_axis_name="c", subcore_axis_name="s",
                                  num_cores=2, num_subcores=16)
    return pl.kernel(
        _mp_body,
        out_type=jax.ShapeDtypeStruct((NPAD, CP), F32),
        mesh=mesh,
        scratch_types=[
            pltpu.VMEM((EBLK,), jnp.int32),
            pltpu.VMEM((EBLK,), jnp.int32),
            pltpu.VMEM((CHUNK,), jnp.int32),
            pltpu.VMEM((CHUNK, CP), F32),
            pltpu.VMEM((CHUNK, CP), F32),
            pltpu.VMEM((CHUNK, CP), F32),
            pltpu.VMEM((CHUNK, CP), F32),
            pltpu.VMEM_SHARED((RNG + TPC, CP), F32),
            pltpu.SemaphoreType.DMA,
            pltpu.SemaphoreType.DMA,
            pltpu.SemaphoreType.DMA,
            pltpu.SemaphoreType.DMA,
        ],
    )(dst1, src1, ew, h)


def kernel(x, edge_index, edge_attr, batch, params):
    n = x.shape[0]
    e = edge_attr.shape[0]

    src = edge_index[0].astype(jnp.int32)
    dst = edge_index[1].astype(jnp.int32)
    src1 = jnp.pad(src, (0, EPAD - e))
    dst1 = jnp.pad(dst, (0, EPAD - e), constant_values=NPAD - 1)
    xp = jnp.pad(x.astype(jnp.int32), (0, NPAD - n))
    bp = jnp.pad(batch.astype(jnp.int32), (0, NPAD - n), constant_values=NB)
    eap = jnp.pad(edge_attr, ((0, EPAD - e), (0, 0)))

    embp = jnp.zeros((128, CP), F32).at[:119, :92].set(params['emb'])

    w1s = jnp.zeros((4, 4, CP), F32)
    b1s = jnp.zeros((4, CP), F32)
    w2s = jnp.zeros((4, CP, CP), F32)
    b2s = jnp.zeros((4, CP), F32)
    nws, nbs = [], []
    for l, p in enumerate(params['convs']):
        d = p['eW1'].shape[1]
        w1s = w1s.at[l, :, :d].set(p['eW1'])
        b1s = b1s.at[l, :d].set(p['eb1'])
        w2s = w2s.at[l, :d, :d].set(p['eW2'])
        b2s = b2s.at[l, :d].set(p['eb2'])
        nws.append(jnp.zeros((CP, CP), F32).at[:d, :HID].set(p['nW']))
        nbs.append(jnp.zeros((1, CP), F32).at[0, :HID].set(p['nb']))

    r = 1024
    h = pl.pallas_call(
        functools.partial(_emb_body, r),
        grid=(NPAD // r,),
        in_specs=[pl.BlockSpec((1, r, 1), lambda i: (i, 0, 0)),
                  pl.BlockSpec((128, CP), lambda i: (0, 0))],
        out_specs=pl.BlockSpec((r, CP), lambda i: (i, 0)),
        out_shape=jax.ShapeDtypeStruct((NPAD, CP), F32),
    )(xp.reshape(NPAD // r, r, 1), embp)

    be = 2048
    ews = pl.pallas_call(
        _edge_body,
        grid=(EPAD // be,),
        in_specs=[pl.BlockSpec((be, 4), lambda i: (i, 0)),
                  pl.BlockSpec((4, 4, CP), lambda i: (0, 0, 0)),
                  pl.BlockSpec((4, CP), lambda i: (0, 0)),
                  pl.BlockSpec((4, CP, CP), lambda i: (0, 0, 0)),
                  pl.BlockSpec((4, CP), lambda i: (0, 0))],
        out_specs=[pl.BlockSpec((be, CP), lambda i: (i, 0))] * 4,
        out_shape=[jax.ShapeDtypeStruct((EPAD, CP), F32)] * 4,
    )(eap, w1s, b1s, w2s, b2s)

    bn = 2560
    for l in range(4):
        agg = _message_pass(dst1, src1, ews[l], h)
        if l < 3:
            h = pl.pallas_call(
                _node_body,
                grid=(NPAD // bn,),
                in_specs=[pl.BlockSpec((bn, CP), lambda i: (i, 0)),
                          pl.BlockSpec((CP, CP), lambda i: (0, 0)),
                          pl.BlockSpec((1, CP), lambda i: (0, 0))],
                out_specs=pl.BlockSpec((bn, CP), lambda i: (i, 0)),
                out_shape=jax.ShapeDtypeStruct((NPAD, CP), F32),
            )(agg, nws[l], nbs[l])
        else:
            sums, cnts = pl.pallas_call(
                functools.partial(_node_pool_body, bn),
                grid=(NPAD // bn,),
                in_specs=[pl.BlockSpec((bn, CP), lambda i: (i, 0)),
                          pl.BlockSpec((CP, CP), lambda i: (0, 0)),
                          pl.BlockSpec((1, CP), lambda i: (0, 0)),
                          pl.BlockSpec((1, bn, 1), lambda i: (i, 0, 0))],
                out_specs=[pl.BlockSpec((NB, CP), lambda i: (0, 0))] * 2,
                out_shape=[jax.ShapeDtypeStruct((NB, CP), F32)] * 2,
            )(agg, nws[l], nbs[l], bp.reshape(NPAD // bn, bn, 1))

    w1p = jnp.zeros((CP, CP), F32).at[:HID, :HID].set(params['rW1'])
    b1p = jnp.zeros((1, CP), F32).at[0, :HID].set(params['rb1'])
    w2p = jnp.zeros((CP, CP), F32).at[:HID, :HID // 2].set(params['rW2'])
    b2p = jnp.zeros((1, CP), F32).at[0, :HID // 2].set(params['rb2'])
    w3p = jnp.zeros((CP, 8), F32).at[:HID // 2, 0].set(params['rW3'][:, 0])
    b3p = jnp.zeros((1, 8), F32).at[0, 0].set(params['rb3'][0])
    out8 = pl.pallas_call(
        _reg_body,
        in_specs=[pl.BlockSpec((NB, CP), lambda: (0, 0)),
                  pl.BlockSpec((NB, CP), lambda: (0, 0)),
                  pl.BlockSpec((CP, CP), lambda: (0, 0)),
                  pl.BlockSpec((1, CP), lambda: (0, 0)),
                  pl.BlockSpec((CP, CP), lambda: (0, 0)),
                  pl.BlockSpec((1, CP), lambda: (0, 0)),
                  pl.BlockSpec((CP, 8), lambda: (0, 0)),
                  pl.BlockSpec((1, 8), lambda: (0, 0))],
        out_specs=pl.BlockSpec((NB, 8), lambda: (0, 0)),
        out_shape=jax.ShapeDtypeStruct((NB, 8), F32),
    )(sums, cnts, w1p, b1p, w2p, b2p, w3p, b3p)
    return out8[:, 0]
